# baseline (speedup 1.0000x reference)
import functools

import jax
import jax.numpy as jnp
from jax import lax
from jax.experimental import pallas as pl
from jax.experimental.pallas import tpu as pltpu
from jax.experimental.pallas import tpu_sc as plsc

_NW = 32
_CHUNK = 128
_SUB = 64


def _sc_gather(user2d, item2d, E_um, E_im, *, batch):
    bpw = batch // _NW
    nch = bpw // _CHUNK
    nsub = bpw // _SUB
    fm = E_um.shape[1]

    mesh = plsc.VectorSubcoreMesh(core_axis_name="c", subcore_axis_name="s")

    @functools.partial(
        pl.kernel,
        mesh=mesh,
        out_type=[
            jax.ShapeDtypeStruct((batch, fm), jnp.float32),
            jax.ShapeDtypeStruct((batch, fm), jnp.float32),
        ],
        scratch_types=[
            pltpu.VMEM((nch, _CHUNK), jnp.int32),
            pltpu.VMEM((nch, _CHUNK), jnp.int32),
            pltpu.VMEM((3, _SUB, fm), jnp.float32),
            pltpu.VMEM((3, _SUB, fm), jnp.float32),
            pltpu.SemaphoreType.DMA,
            pltpu.SemaphoreType.DMA,
        ],
    )
    def k(user_h, item_h, eum_h, eim_h, out_mu, out_mi,
          idx_u, idx_i, buf_mu, buf_mi, semg, semw):
        wid = lax.axis_index("s") * 2 + lax.axis_index("c")
        base = wid * bpw
        pltpu.sync_copy(user_h.at[pl.ds(wid * nch, nch)], idx_u)
        pltpu.sync_copy(item_h.at[pl.ds(wid * nch, nch)], idx_i)
        def idx_slice(idx, j):
            return idx.at[j // 2, pl.ds((j % 2) * _SUB, _SUB)]

        gh = {}
        wh = {}
        for j in range(nsub):
            b = j % 3
            if j >= 3:
                for hh in wh[j - 3]:
                    hh.wait()
            gh[j] = [
                pltpu.async_copy(eum_h.at[idx_slice(idx_u, j)],
                                 buf_mu.at[b], semg),
                pltpu.async_copy(eim_h.at[idx_slice(idx_i, j)],
                                 buf_mi.at[b], semg),
            ]
            if j >= 1:
                for hh in gh[j - 1]:
                    hh.wait()
                out_sl = pl.ds(base + (j - 1) * _SUB, _SUB)
                bb = (j - 1) % 3
                wh[j - 1] = [
                    pltpu.async_copy(buf_mu.at[bb], out_mu.at[out_sl], semw),
                    pltpu.async_copy(buf_mi.at[bb], out_mi.at[out_sl], semw),
                ]
        j = nsub - 1
        for hh in gh[j]:
            hh.wait()
        out_sl = pl.ds(base + j * _SUB, _SUB)
        wh[j] = [
            pltpu.async_copy(buf_mu.at[j % 3], out_mu.at[out_sl], semw),
            pltpu.async_copy(buf_mi.at[j % 3], out_mi.at[out_sl], semw),
        ]
        for jj in range(max(0, nsub - 3), nsub):
            for hh in wh[jj]:
                hh.wait()

    return k(user2d, item2d, E_um, E_im)


def _tc_mlp(mu, mi, W0a, W0b, b0, W1, b1, W2, b2, Wpb, bp, *, batch, tb):
    grid = (batch // tb,)
    rows = tb // 128

    def body(mu_ref, mi_ref, w0a, w0b, b0r, w1, b1r, w2, b2r, wpb, bpr,
             out_ref):
        f32 = jnp.float32
        h = jnp.dot(mu_ref[...], w0a[...], preferred_element_type=f32)
        h += jnp.dot(mi_ref[...], w0b[...], preferred_element_type=f32)
        h = jnp.maximum(h + b0r[...], 0.0)
        h = jnp.maximum(jnp.dot(h, w1[...], preferred_element_type=f32) + b1r[...], 0.0)
        h = jnp.maximum(jnp.dot(h, w2[...], preferred_element_type=f32) + b2r[...], 0.0)
        zm = jnp.sum(h * wpb[...], axis=1) + bpr[0, 0]
        out_ref[...] = zm.reshape(rows, 128)

    full = lambda shape: pl.BlockSpec(shape, lambda i: (0,) * len(shape))
    row = lambda shape: pl.BlockSpec(shape, lambda i: (i,) + (0,) * (len(shape) - 1))
    return pl.pallas_call(
        body,
        grid=grid,
        in_specs=[
            row((tb, mu.shape[1])), row((tb, mi.shape[1])),
            full(W0a.shape), full(W0b.shape), full(b0.shape),
            full(W1.shape), full(b1.shape),
            full(W2.shape), full(b2.shape),
            full((1, W0a.shape[1] // 4)), full((1, 1)),
        ],
        out_specs=row((rows, 128)),
        out_shape=jax.ShapeDtypeStruct((batch // 128, 128), jnp.float32),
        compiler_params=pltpu.CompilerParams(
            dimension_semantics=("arbitrary",)),
    )(mu, mi, W0a, W0b, b0, W1, b1, W2, b2, Wpb, bp)


def _tc_gmf(eu, ei, zmlp, Wpa, *, batch, tb, fg):
    grid = (batch // tb,)
    rows = tb // 128

    def body(eu_ref, ei_ref, z_ref, wpa, out_ref):
        f32 = jnp.float32
        gmfT = eu_ref[...] * ei_ref[...]
        zgT = jnp.dot(wpa[...], gmfT, preferred_element_type=f32)
        z = z_ref[...] + zgT.reshape(rows, 128)
        out_ref[...] = 1.0 / (1.0 + jnp.exp(-z))

    return pl.pallas_call(
        body,
        grid=grid,
        in_specs=[
            pl.BlockSpec((fg, tb), lambda i: (0, i)),
            pl.BlockSpec((fg, tb), lambda i: (0, i)),
            pl.BlockSpec((rows, 128), lambda i: (i, 0)),
            pl.BlockSpec((1, fg), lambda i: (0, 0)),
        ],
        out_specs=pl.BlockSpec((rows, 128), lambda i: (i, 0)),
        out_shape=jax.ShapeDtypeStruct((batch // 128, 128), jnp.float32),
        compiler_params=pltpu.CompilerParams(
            dimension_semantics=("arbitrary",)),
    )(eu, ei, zmlp, Wpa)


def kernel(user, item, E_ug, E_ig, E_um, E_im, W0, b0, W1, b1, W2, b2, Wp, bp):
    batch = user.shape[0]
    fm = E_um.shape[1]
    fg = E_ug.shape[1]
    user2d = user.reshape(batch // _CHUNK, _CHUNK)
    item2d = item.reshape(batch // _CHUNK, _CHUNK)
    mu, mi = _sc_gather(user2d, item2d, E_um, E_im, batch=batch)
    E_ug2, E_ig2, user3, item3, mu2, mi2 = lax.optimization_barrier(
        (E_ug, E_ig, user, item, mu, mi))
    eu = E_ug2.at[user3].get(mode="promise_in_bounds").T
    ei = E_ig2.at[item3].get(mode="promise_in_bounds").T
    W0a, W0b = W0[:fm, :], W0[fm:, :]
    Wpa, Wpb = Wp[:fg, :].reshape(1, fg), Wp[fg:, :].reshape(1, fg)
    zmlp = _tc_mlp(mu2, mi2,
                   W0a, W0b, b0.reshape(1, -1),
                   W1, b1.reshape(1, -1), W2, b2.reshape(1, -1),
                   Wpb, bp.reshape(1, 1),
                   batch=batch, tb=2048)
    pred = _tc_gmf(eu, ei, zmlp, Wpa, batch=batch, tb=8192, fg=fg)
    return pred.reshape(-1)


# speedup vs baseline: 1.0073x; 1.0073x over previous
import functools

import jax
import jax.numpy as jnp
from jax import lax
from jax.experimental import pallas as pl
from jax.experimental.pallas import tpu as pltpu
from jax.experimental.pallas import tpu_sc as plsc

_NW = 32
_CHUNK = 128
_SUB = 128


def _sc_gather(user2d, item2d, E_um, E_im, *, batch):
    bpw = batch // _NW
    nch = bpw // _CHUNK
    nsub = bpw // _SUB
    fm = E_um.shape[1]

    mesh = plsc.VectorSubcoreMesh(core_axis_name="c", subcore_axis_name="s")

    @functools.partial(
        pl.kernel,
        mesh=mesh,
        out_type=[
            jax.ShapeDtypeStruct((batch, fm), jnp.float32),
            jax.ShapeDtypeStruct((batch, fm), jnp.float32),
        ],
        scratch_types=[
            pltpu.VMEM((nch, _CHUNK), jnp.int32),
            pltpu.VMEM((nch, _CHUNK), jnp.int32),
            pltpu.VMEM((3, _SUB, fm), jnp.float32),
            pltpu.VMEM((3, _SUB, fm), jnp.float32),
            pltpu.SemaphoreType.DMA,
            pltpu.SemaphoreType.DMA,
        ],
    )
    def k(user_h, item_h, eum_h, eim_h, out_mu, out_mi,
          idx_u, idx_i, buf_mu, buf_mi, semg, semw):
        wid = lax.axis_index("s") * 2 + lax.axis_index("c")
        base = wid * bpw
        pltpu.sync_copy(user_h.at[pl.ds(wid * nch, nch)], idx_u)
        pltpu.sync_copy(item_h.at[pl.ds(wid * nch, nch)], idx_i)
        per = _CHUNK // _SUB

        def idx_slice(idx, j):
            return idx.at[j // per, pl.ds((j % per) * _SUB, _SUB)]

        gh = {}
        wh = {}
        for j in range(nsub):
            b = j % 3
            if j >= 3:
                for hh in wh[j - 3]:
                    hh.wait()
            gh[j] = [
                pltpu.async_copy(eum_h.at[idx_slice(idx_u, j)],
                                 buf_mu.at[b], semg),
                pltpu.async_copy(eim_h.at[idx_slice(idx_i, j)],
                                 buf_mi.at[b], semg),
            ]
            if j >= 1:
                for hh in gh[j - 1]:
                    hh.wait()
                out_sl = pl.ds(base + (j - 1) * _SUB, _SUB)
                bb = (j - 1) % 3
                wh[j - 1] = [
                    pltpu.async_copy(buf_mu.at[bb], out_mu.at[out_sl], semw),
                    pltpu.async_copy(buf_mi.at[bb], out_mi.at[out_sl], semw),
                ]
        j = nsub - 1
        for hh in gh[j]:
            hh.wait()
        out_sl = pl.ds(base + j * _SUB, _SUB)
        wh[j] = [
            pltpu.async_copy(buf_mu.at[j % 3], out_mu.at[out_sl], semw),
            pltpu.async_copy(buf_mi.at[j % 3], out_mi.at[out_sl], semw),
        ]
        for jj in range(max(0, nsub - 3), nsub):
            for hh in wh[jj]:
                hh.wait()

    return k(user2d, item2d, E_um, E_im)


def _tc_mlp(mu, mi, W0a, W0b, b0, W1, b1, W2, b2, Wpb, bp, *, batch, tb):
    grid = (batch // tb,)
    rows = tb // 128

    def body(mu_ref, mi_ref, w0a, w0b, b0r, w1, b1r, w2, b2r, wpb, bpr,
             out_ref):
        f32 = jnp.float32
        h = jnp.dot(mu_ref[...], w0a[...], preferred_element_type=f32)
        h += jnp.dot(mi_ref[...], w0b[...], preferred_element_type=f32)
        h = jnp.maximum(h + b0r[...], 0.0)
        h = jnp.maximum(jnp.dot(h, w1[...], preferred_element_type=f32) + b1r[...], 0.0)
        h = jnp.maximum(jnp.dot(h, w2[...], preferred_element_type=f32) + b2r[...], 0.0)
        zm = jnp.sum(h * wpb[...], axis=1) + bpr[0, 0]
        out_ref[...] = zm.reshape(rows, 128)

    full = lambda shape: pl.BlockSpec(shape, lambda i: (0,) * len(shape))
    row = lambda shape: pl.BlockSpec(shape, lambda i: (i,) + (0,) * (len(shape) - 1))
    return pl.pallas_call(
        body,
        grid=grid,
        in_specs=[
            row((tb, mu.shape[1])), row((tb, mi.shape[1])),
            full(W0a.shape), full(W0b.shape), full(b0.shape),
            full(W1.shape), full(b1.shape),
            full(W2.shape), full(b2.shape),
            full((1, W0a.shape[1] // 4)), full((1, 1)),
        ],
        out_specs=row((rows, 128)),
        out_shape=jax.ShapeDtypeStruct((batch // 128, 128), jnp.float32),
        compiler_params=pltpu.CompilerParams(
            dimension_semantics=("arbitrary",)),
    )(mu, mi, W0a, W0b, b0, W1, b1, W2, b2, Wpb, bp)


def _tc_gmf(eu, ei, zmlp, Wpa, *, batch, tb, fg):
    grid = (batch // tb,)
    rows = tb // 128

    def body(eu_ref, ei_ref, z_ref, wpa, out_ref):
        f32 = jnp.float32
        gmfT = eu_ref[...] * ei_ref[...]
        zgT = jnp.dot(wpa[...], gmfT, preferred_element_type=f32)
        z = z_ref[...] + zgT.reshape(rows, 128)
        out_ref[...] = 1.0 / (1.0 + jnp.exp(-z))

    return pl.pallas_call(
        body,
        grid=grid,
        in_specs=[
            pl.BlockSpec((fg, tb), lambda i: (0, i)),
            pl.BlockSpec((fg, tb), lambda i: (0, i)),
            pl.BlockSpec((rows, 128), lambda i: (i, 0)),
            pl.BlockSpec((1, fg), lambda i: (0, 0)),
        ],
        out_specs=pl.BlockSpec((rows, 128), lambda i: (i, 0)),
        out_shape=jax.ShapeDtypeStruct((batch // 128, 128), jnp.float32),
        compiler_params=pltpu.CompilerParams(
            dimension_semantics=("arbitrary",)),
    )(eu, ei, zmlp, Wpa)


def kernel(user, item, E_ug, E_ig, E_um, E_im, W0, b0, W1, b1, W2, b2, Wp, bp):
    batch = user.shape[0]
    fm = E_um.shape[1]
    fg = E_ug.shape[1]
    user2d = user.reshape(batch // _CHUNK, _CHUNK)
    item2d = item.reshape(batch // _CHUNK, _CHUNK)
    mu, mi = _sc_gather(user2d, item2d, E_um, E_im, batch=batch)
    E_ug2, E_ig2, user3, item3, mu2, mi2 = lax.optimization_barrier(
        (E_ug, E_ig, user, item, mu, mi))
    eu = E_ug2.at[user3].get(mode="promise_in_bounds").T
    ei = E_ig2.at[item3].get(mode="promise_in_bounds").T
    W0a, W0b = W0[:fm, :], W0[fm:, :]
    Wpa, Wpb = Wp[:fg, :].reshape(1, fg), Wp[fg:, :].reshape(1, fg)
    zmlp = _tc_mlp(mu2, mi2,
                   W0a, W0b, b0.reshape(1, -1),
                   W1, b1.reshape(1, -1), W2, b2.reshape(1, -1),
                   Wpb, bp.reshape(1, 1),
                   batch=batch, tb=2048)
    pred = _tc_gmf(eu, ei, zmlp, Wpa, batch=batch, tb=8192, fg=fg)
    return pred.reshape(-1)


# speedup vs baseline: 1.0093x; 1.0020x over previous
import functools

import jax
import jax.numpy as jnp
from jax import lax
from jax.experimental import pallas as pl
from jax.experimental.pallas import tpu as pltpu
from jax.experimental.pallas import tpu_sc as plsc

_NW = 32
_CHUNK = 128
_SUB = 128


def _sc_gather(user2d, item2d, E_um, E_im, *, batch):
    bpw = batch // _NW
    nch = bpw // _CHUNK
    nsub = bpw // _SUB
    fm = E_um.shape[1]

    mesh = plsc.VectorSubcoreMesh(core_axis_name="c", subcore_axis_name="s")

    @functools.partial(
        pl.kernel,
        mesh=mesh,
        out_type=[
            jax.ShapeDtypeStruct((batch, fm), jnp.float32),
            jax.ShapeDtypeStruct((batch, fm), jnp.float32),
        ],
        scratch_types=[
            pltpu.VMEM((nch, _CHUNK), jnp.int32),
            pltpu.VMEM((nch, _CHUNK), jnp.int32),
            pltpu.VMEM((3, _SUB, fm), jnp.float32),
            pltpu.VMEM((3, _SUB, fm), jnp.float32),
            pltpu.SemaphoreType.DMA,
            pltpu.SemaphoreType.DMA,
        ],
    )
    def k(user_h, item_h, eum_h, eim_h, out_mu, out_mi,
          idx_u, idx_i, buf_mu, buf_mi, semg, semw):
        wid = lax.axis_index("s") * 2 + lax.axis_index("c")
        base = wid * bpw
        pltpu.sync_copy(user_h.at[pl.ds(wid * nch, nch)], idx_u)
        pltpu.sync_copy(item_h.at[pl.ds(wid * nch, nch)], idx_i)
        per = _CHUNK // _SUB

        def idx_slice(idx, j):
            return idx.at[j // per, pl.ds((j % per) * _SUB, _SUB)]

        gh = {}
        wh = {}
        for j in range(nsub):
            b = j % 3
            if j >= 3:
                for hh in wh[j - 3]:
                    hh.wait()
            gh[j] = [
                pltpu.async_copy(eum_h.at[idx_slice(idx_u, j)],
                                 buf_mu.at[b], semg),
                pltpu.async_copy(eim_h.at[idx_slice(idx_i, j)],
                                 buf_mi.at[b], semg),
            ]
            if j >= 1:
                for hh in gh[j - 1]:
                    hh.wait()
                out_sl = pl.ds(base + (j - 1) * _SUB, _SUB)
                bb = (j - 1) % 3
                wh[j - 1] = [
                    pltpu.async_copy(buf_mu.at[bb], out_mu.at[out_sl], semw),
                    pltpu.async_copy(buf_mi.at[bb], out_mi.at[out_sl], semw),
                ]
        j = nsub - 1
        for hh in gh[j]:
            hh.wait()
        out_sl = pl.ds(base + j * _SUB, _SUB)
        wh[j] = [
            pltpu.async_copy(buf_mu.at[j % 3], out_mu.at[out_sl], semw),
            pltpu.async_copy(buf_mi.at[j % 3], out_mi.at[out_sl], semw),
        ]
        for jj in range(max(0, nsub - 3), nsub):
            for hh in wh[jj]:
                hh.wait()

    return k(user2d, item2d, E_um, E_im)


def _tc_mlp(mu, mi, W0a, W0b, b0, W1, b1, W2, b2, Wpb, bp, *, batch, tb):
    grid = (batch // tb,)
    rows = tb // 128

    def body(mu_ref, mi_ref, w0a, w0b, b0r, w1, b1r, w2, b2r, wpb, bpr,
             out_ref):
        f32 = jnp.float32
        h = jnp.dot(mu_ref[...], w0a[...], preferred_element_type=f32)
        h += jnp.dot(mi_ref[...], w0b[...], preferred_element_type=f32)
        h = jnp.maximum(h + b0r[...], 0.0)
        h = jnp.maximum(jnp.dot(h, w1[...], preferred_element_type=f32) + b1r[...], 0.0)
        h = jnp.maximum(jnp.dot(h, w2[...], preferred_element_type=f32) + b2r[...], 0.0)
        zm = jnp.sum(h * wpb[...], axis=1) + bpr[0, 0]
        out_ref[...] = zm.reshape(rows, 128)

    full = lambda shape: pl.BlockSpec(shape, lambda i: (0,) * len(shape))
    row = lambda shape: pl.BlockSpec(shape, lambda i: (i,) + (0,) * (len(shape) - 1))
    return pl.pallas_call(
        body,
        grid=grid,
        in_specs=[
            row((tb, mu.shape[1])), row((tb, mi.shape[1])),
            full(W0a.shape), full(W0b.shape), full(b0.shape),
            full(W1.shape), full(b1.shape),
            full(W2.shape), full(b2.shape),
            full(Wpb.shape), full((1, 1)),
        ],
        out_specs=row((rows, 128)),
        out_shape=jax.ShapeDtypeStruct((batch // 128, 128), jnp.float32),
        compiler_params=pltpu.CompilerParams(
            dimension_semantics=("arbitrary",)),
    )(mu, mi, W0a, W0b, b0, W1, b1, W2, b2, Wpb, bp)


def _tc_gmf(eu, ei, zmlp, Wpa, *, batch, tb, fg):
    grid = (batch // tb,)
    rows = tb // 128

    def body(eu_ref, ei_ref, z_ref, wpa, out_ref):
        f32 = jnp.float32
        gmfT = eu_ref[...] * ei_ref[...]
        zgT = jnp.dot(wpa[...], gmfT, preferred_element_type=f32)
        z = z_ref[...] + zgT.reshape(rows, 128)
        out_ref[...] = 1.0 / (1.0 + jnp.exp(-z))

    return pl.pallas_call(
        body,
        grid=grid,
        in_specs=[
            pl.BlockSpec((fg, tb), lambda i: (0, i)),
            pl.BlockSpec((fg, tb), lambda i: (0, i)),
            pl.BlockSpec((rows, 128), lambda i: (i, 0)),
            pl.BlockSpec((1, fg), lambda i: (0, 0)),
        ],
        out_specs=pl.BlockSpec((rows, 128), lambda i: (i, 0)),
        out_shape=jax.ShapeDtypeStruct((batch // 128, 128), jnp.float32),
        compiler_params=pltpu.CompilerParams(
            dimension_semantics=("arbitrary",)),
    )(eu, ei, zmlp, Wpa)


def kernel(user, item, E_ug, E_ig, E_um, E_im, W0, b0, W1, b1, W2, b2, Wp, bp):
    batch = user.shape[0]
    fm = E_um.shape[1]
    fg = E_ug.shape[1]
    user2d = user.reshape(batch // _CHUNK, _CHUNK)
    item2d = item.reshape(batch // _CHUNK, _CHUNK)
    mu, mi = _sc_gather(user2d, item2d, E_um, E_im, batch=batch)
    E_ug2, E_ig2, user3, item3, mu2, mi2 = lax.optimization_barrier(
        (E_ug, E_ig, user, item, mu, mi))
    eu = E_ug2.at[user3].get(mode="promise_in_bounds").T
    ei = E_ig2.at[item3].get(mode="promise_in_bounds").T
    W0a, W0b = W0[:fm, :], W0[fm:, :]
    Wpa, Wpb = Wp[:fg, :].reshape(1, fg), Wp[fg:, :].reshape(1, fg)
    zmlp = _tc_mlp(mu2, mi2,
                   W0a, W0b, b0.reshape(1, -1),
                   W1, b1.reshape(1, -1), W2, b2.reshape(1, -1),
                   Wpb, bp.reshape(1, 1),
                   batch=batch, tb=2048)
    pred = _tc_gmf(eu, ei, zmlp, Wpa, batch=batch, tb=8192, fg=fg)
    return pred.reshape(-1)
